# baseline (speedup 1.0000x reference)
import jax
import jax.numpy as jnp
from jax.experimental import pallas as pl

BS, M, P, DIM = 32, 128, 20, 128
N = BS * M
BLK = 256
BN_SCALE = 0.9999950000374997


def _fused_kernel(x_ref, mask_ref, t_ref, r_ref, tl_ref, has_ref, s_ref,
                  w1_ref, b1_ref, w2_ref, b2_ref, w3_ref, b3_ref,
                  w4_ref, b4_ref, slw1_ref, slb1_ref, slw2_ref, slb2_ref,
                  temb_ref, remb_ref, tlemb_ref, uemb_ref, o_ref):
    f32 = jnp.float32
    bf16 = jnp.bfloat16
    bn = f32(BN_SCALE)

    w1 = w1_ref[...]
    b1 = b1_ref[...]
    w2 = (w2_ref[...] * bn).astype(bf16)
    b2 = b2_ref[...]
    mask = mask_ref[...].astype(f32)

    h2s = []
    pooled = None
    for p in range(P):
        x = x_ref[:, 8 * p:8 * p + 6]
        h1 = jax.nn.relu(jnp.dot(x, w1, preferred_element_type=f32) + b1)
        h2 = jnp.dot(h1.astype(bf16), w2, preferred_element_type=f32) + b2
        h2 = (h2 * mask[:, p:p + 1]).astype(bf16)
        h2s.append(h2)
        pooled = h2 if pooled is None else jnp.maximum(pooled, h2)

    w3t = w3_ref[0:256].astype(bf16)
    w3b = w3_ref[256:512].astype(bf16)
    pb = jnp.dot(pooled, w3b, preferred_element_type=f32) + b3_ref[...]

    w4 = (w4_ref[...] * bn).astype(bf16)
    b4 = b4_ref[...]
    out = None
    for p in range(P):
        g1 = jax.nn.relu(jnp.dot(h2s[p], w3t, preferred_element_type=f32) + pb)
        g = jnp.dot(g1.astype(bf16), w4, preferred_element_type=f32) + b4
        g = g * mask[:, p:p + 1]
        out = g if out is None else jnp.maximum(out, g)

    has = has_ref[...].astype(f32)
    iota = jax.lax.broadcasted_iota(jnp.int32, (BLK, 16), 1)
    onehot = ((iota == t_ref[...]).astype(f32)
              + (iota == r_ref[...] + 3).astype(f32)
              + (iota == tl_ref[...] + 5).astype(f32)
              + (iota == 9).astype(f32) * (1.0 - has))
    emb = jnp.concatenate(
        [temb_ref[...], remb_ref[...], tlemb_ref[...], uemb_ref[...],
         jnp.zeros((6, DIM), f32)], axis=0)
    cat = jnp.dot(onehot, emb, preferred_element_type=f32)

    hs = jax.nn.relu(s_ref[...] * slw1_ref[...] + slb1_ref[...])
    sl = jnp.dot(hs, slw2_ref[...], preferred_element_type=f32) + slb2_ref[...]
    o_ref[...] = out + cat + sl * has


def kernel(polygon_center, polygon_type, polygon_on_route, polygon_tl_status,
           polygon_has_speed_limit, polygon_speed_limit, point_position,
           point_vector, point_orientation, polygon_orientation, valid_mask,
           pe_W1, pe_b1, pe_W2, pe_b2, pe_W3, pe_b3, pe_W4, pe_b4,
           sl_W1, sl_b1, sl_W2, sl_b2, type_emb, on_route_emb, tl_emb, unk_emb):
    f32 = jnp.float32
    pp = point_position[:, :, 0].reshape(N, P, 2)
    pv = point_vector[:, :, 0].reshape(N, P, 2)
    po = point_orientation[:, :, 0].reshape(N, P)
    c2 = polygon_center[..., :2].reshape(N, 1, 2)
    feat = jnp.concatenate(
        [pp - c2, pv, jnp.cos(po)[..., None], jnp.sin(po)[..., None],
         jnp.zeros((N, P, 2), f32)], axis=-1)
    x = feat.reshape(N, P * 8)

    grid = (N // BLK,)
    row = lambda shape: pl.BlockSpec(shape, lambda i: (i, 0))
    rep = lambda shape: pl.BlockSpec(shape, lambda i: (0, 0))
    out = pl.pallas_call(
        _fused_kernel,
        grid=grid,
        in_specs=[
            row((BLK, P * 8)), row((BLK, P)),
            row((BLK, 1)), row((BLK, 1)), row((BLK, 1)), row((BLK, 1)),
            row((BLK, 1)),
            rep((6, 128)), rep((1, 128)),
            rep((128, 256)), rep((1, 256)),
            rep((512, 256)), rep((1, 256)),
            rep((256, 128)), rep((1, 128)),
            rep((1, 128)), rep((1, 128)), rep((128, 128)), rep((1, 128)),
            rep((3, 128)), rep((2, 128)), rep((4, 128)), rep((1, 128)),
        ],
        out_specs=pl.BlockSpec((BLK, DIM), lambda i: (i, 0)),
        out_shape=jax.ShapeDtypeStruct((N, DIM), f32),
    )(x, valid_mask.reshape(N, P),
      polygon_type.reshape(N, 1), polygon_on_route.reshape(N, 1),
      polygon_tl_status.reshape(N, 1), polygon_has_speed_limit.reshape(N, 1),
      polygon_speed_limit.reshape(N, 1),
      pe_W1, pe_b1.reshape(1, 128), pe_W2, pe_b2.reshape(1, 256),
      pe_W3, pe_b3.reshape(1, 256), pe_W4, pe_b4.reshape(1, 128),
      sl_W1, sl_b1.reshape(1, 128), sl_W2, sl_b2.reshape(1, 128),
      type_emb, on_route_emb, tl_emb, unk_emb)
    return out.reshape(BS, M, DIM)


# speedup vs baseline: 1.0060x; 1.0060x over previous
import jax
import jax.numpy as jnp
from jax.experimental import pallas as pl
from jax.experimental.pallas import tpu as pltpu

BS, M, P, DIM = 32, 128, 20, 128
N = BS * M
BLK = 256
BN_SCALE = 0.9999950000374997


def _fused_kernel(x_ref, mask_ref, t_ref, r_ref, tl_ref, has_ref, s_ref,
                  w1_ref, b1_ref, w2_ref, b2_ref, w3_ref, b3_ref,
                  w4_ref, b4_ref, slw1_ref, slb1_ref, slw2_ref, slb2_ref,
                  temb_ref, remb_ref, tlemb_ref, uemb_ref, o_ref,
                  w2s_ref, w3ts_ref, w3bs_ref, w4s_ref):
    f32 = jnp.float32
    bf16 = jnp.bfloat16
    bn = f32(BN_SCALE)

    @pl.when(pl.program_id(0) == 0)
    def _prep():
        w2s_ref[...] = (w2_ref[...] * bn).astype(bf16)
        w3ts_ref[...] = w3_ref[0:256].astype(bf16)
        w3bs_ref[...] = w3_ref[256:512].astype(bf16)
        w4s_ref[...] = (w4_ref[...] * bn).astype(bf16)

    w1 = w1_ref[...]
    b1 = b1_ref[...]
    w2 = w2s_ref[...]
    b2 = b2_ref[...]
    mask = mask_ref[...].astype(f32)

    h2s = []
    pooled = None
    for p in range(P):
        x = x_ref[:, 8 * p:8 * p + 6]
        h1 = jax.nn.relu(jnp.dot(x, w1, preferred_element_type=f32) + b1)
        h2 = jnp.dot(h1.astype(bf16), w2, preferred_element_type=f32) + b2
        h2 = (h2 * mask[:, p:p + 1]).astype(bf16)
        h2s.append(h2)
        pooled = h2 if pooled is None else jnp.maximum(pooled, h2)

    w3t = w3ts_ref[...]
    pb = jnp.dot(pooled, w3bs_ref[...], preferred_element_type=f32) + b3_ref[...]

    w4 = w4s_ref[...]
    b4 = b4_ref[...]
    out = None
    for p in range(P):
        g1 = jax.nn.relu(jnp.dot(h2s[p], w3t, preferred_element_type=f32) + pb)
        g = jnp.dot(g1.astype(bf16), w4, preferred_element_type=f32) + b4
        g = g * mask[:, p:p + 1]
        out = g if out is None else jnp.maximum(out, g)

    has = has_ref[...].astype(f32)
    iota = jax.lax.broadcasted_iota(jnp.int32, (BLK, 16), 1)
    onehot = ((iota == t_ref[...]).astype(f32)
              + (iota == r_ref[...] + 3).astype(f32)
              + (iota == tl_ref[...] + 5).astype(f32)
              + (iota == 9).astype(f32) * (1.0 - has))
    emb = jnp.concatenate(
        [temb_ref[...], remb_ref[...], tlemb_ref[...], uemb_ref[...],
         jnp.zeros((6, DIM), f32)], axis=0)
    cat = jnp.dot(onehot, emb, preferred_element_type=f32)

    hs = jax.nn.relu(s_ref[...] * slw1_ref[...] + slb1_ref[...])
    sl = jnp.dot(hs, slw2_ref[...], preferred_element_type=f32) + slb2_ref[...]
    o_ref[...] = out + cat + sl * has


def kernel(polygon_center, polygon_type, polygon_on_route, polygon_tl_status,
           polygon_has_speed_limit, polygon_speed_limit, point_position,
           point_vector, point_orientation, polygon_orientation, valid_mask,
           pe_W1, pe_b1, pe_W2, pe_b2, pe_W3, pe_b3, pe_W4, pe_b4,
           sl_W1, sl_b1, sl_W2, sl_b2, type_emb, on_route_emb, tl_emb, unk_emb):
    f32 = jnp.float32
    pp = point_position[:, :, 0].reshape(N, P, 2)
    pv = point_vector[:, :, 0].reshape(N, P, 2)
    po = point_orientation[:, :, 0].reshape(N, P)
    c2 = polygon_center[..., :2].reshape(N, 1, 2)
    feat = jnp.concatenate(
        [pp - c2, pv, jnp.cos(po)[..., None], jnp.sin(po)[..., None],
         jnp.zeros((N, P, 2), f32)], axis=-1)
    x = feat.reshape(N, P * 8)

    grid = (N // BLK,)
    row = lambda shape: pl.BlockSpec(shape, lambda i: (i, 0))
    rep = lambda shape: pl.BlockSpec(shape, lambda i: (0, 0))
    out = pl.pallas_call(
        _fused_kernel,
        grid=grid,
        in_specs=[
            row((BLK, P * 8)), row((BLK, P)),
            row((BLK, 1)), row((BLK, 1)), row((BLK, 1)), row((BLK, 1)),
            row((BLK, 1)),
            rep((6, 128)), rep((1, 128)),
            rep((128, 256)), rep((1, 256)),
            rep((512, 256)), rep((1, 256)),
            rep((256, 128)), rep((1, 128)),
            rep((1, 128)), rep((1, 128)), rep((128, 128)), rep((1, 128)),
            rep((3, 128)), rep((2, 128)), rep((4, 128)), rep((1, 128)),
        ],
        out_specs=pl.BlockSpec((BLK, DIM), lambda i: (i, 0)),
        out_shape=jax.ShapeDtypeStruct((N, DIM), f32),
        scratch_shapes=[
            pltpu.VMEM((128, 256), jnp.bfloat16),
            pltpu.VMEM((256, 256), jnp.bfloat16),
            pltpu.VMEM((256, 256), jnp.bfloat16),
            pltpu.VMEM((256, 128), jnp.bfloat16),
        ],
    )(x, valid_mask.reshape(N, P),
      polygon_type.reshape(N, 1), polygon_on_route.reshape(N, 1),
      polygon_tl_status.reshape(N, 1), polygon_has_speed_limit.reshape(N, 1),
      polygon_speed_limit.reshape(N, 1),
      pe_W1, pe_b1.reshape(1, 128), pe_W2, pe_b2.reshape(1, 256),
      pe_W3, pe_b3.reshape(1, 256), pe_W4, pe_b4.reshape(1, 128),
      sl_W1, sl_b1.reshape(1, 128), sl_W2, sl_b2.reshape(1, 128),
      type_emb, on_route_emb, tl_emb, unk_emb)
    return out.reshape(BS, M, DIM)


# speedup vs baseline: 1.1334x; 1.1266x over previous
import jax
import jax.numpy as jnp
from jax.experimental import pallas as pl
from jax.experimental.pallas import tpu as pltpu

BS, M, P, DIM = 32, 128, 20, 128
N = BS * M
BLK = 256
BN_SCALE = 0.9999950000374997


def _fused_kernel(pp_ref, pv_ref, po_ref, ctr_ref, mask_ref,
                  t_ref, r_ref, tl_ref, has_ref, s_ref,
                  w1_ref, b1_ref, w2_ref, b2_ref, w3_ref, b3_ref,
                  w4_ref, b4_ref, slw1_ref, slb1_ref, slw2_ref, slb2_ref,
                  temb_ref, remb_ref, tlemb_ref, uemb_ref, o_ref,
                  w1c_ref, w2s_ref, w3ts_ref, w3bs_ref, w4s_ref):
    f32 = jnp.float32
    bf16 = jnp.bfloat16
    bn = f32(BN_SCALE)

    @pl.when(pl.program_id(0) == 0)
    def _prep():
        w2s_ref[...] = (w2_ref[...] * bn).astype(bf16)
        w3ts_ref[...] = w3_ref[0:256].astype(bf16)
        w3bs_ref[...] = w3_ref[256:512].astype(bf16)
        w4s_ref[...] = (w4_ref[...] * bn).astype(bf16)
        w1c_ref[...] = jnp.zeros((128, P * 128), f32)
        for p in range(P):
            c = 128 * p
            w1c_ref[2 * p:2 * p + 2, c:c + 128] = w1_ref[0:2]
            w1c_ref[2 * P + 2 * p:2 * P + 2 * p + 2, c:c + 128] = w1_ref[2:4]
            w1c_ref[4 * P + p:4 * P + p + 1, c:c + 128] = w1_ref[4:5]
            w1c_ref[5 * P + p:5 * P + p + 1, c:c + 128] = w1_ref[5:6]
            w1c_ref[6 * P:6 * P + 2, c:c + 128] = -w1_ref[0:2]

    b1 = b1_ref[...]
    w2 = w2s_ref[...]
    b2 = b2_ref[...]
    mask = mask_ref[...].astype(f32)

    po = po_ref[:, 0:P]
    lhs = jnp.concatenate(
        [pp_ref[:, 0:2 * P], pv_ref[:, 0:2 * P], jnp.cos(po), jnp.sin(po),
         ctr_ref[:, 0:2], jnp.zeros((BLK, 128 - 6 * P - 2), f32)], axis=1)
    h1a = jnp.dot(lhs, w1c_ref[...], preferred_element_type=f32)

    h2s = []
    pooled = None
    for p in range(P):
        h1 = jax.nn.relu(h1a[:, 128 * p:128 * p + 128] + b1)
        h2 = jnp.dot(h1.astype(bf16), w2, preferred_element_type=f32) + b2
        h2 = (h2 * mask[:, p:p + 1]).astype(bf16)
        h2s.append(h2)
        pooled = h2 if pooled is None else jnp.maximum(pooled, h2)

    w3t = w3ts_ref[...]
    pb = jnp.dot(pooled, w3bs_ref[...], preferred_element_type=f32) + b3_ref[...]

    w4 = w4s_ref[...]
    b4 = b4_ref[...]
    out = None
    for p in range(P):
        g1 = jax.nn.relu(jnp.dot(h2s[p], w3t, preferred_element_type=f32) + pb)
        g = jnp.dot(g1.astype(bf16), w4, preferred_element_type=f32) + b4
        g = g * mask[:, p:p + 1]
        out = g if out is None else jnp.maximum(out, g)

    has = has_ref[...].astype(f32)
    iota = jax.lax.broadcasted_iota(jnp.int32, (BLK, 16), 1)
    onehot = ((iota == t_ref[...]).astype(f32)
              + (iota == r_ref[...] + 3).astype(f32)
              + (iota == tl_ref[...] + 5).astype(f32)
              + (iota == 9).astype(f32) * (1.0 - has))
    emb = jnp.concatenate(
        [temb_ref[...], remb_ref[...], tlemb_ref[...], uemb_ref[...],
         jnp.zeros((6, DIM), f32)], axis=0)
    cat = jnp.dot(onehot, emb, preferred_element_type=f32)

    hs = jax.nn.relu(s_ref[...] * slw1_ref[...] + slb1_ref[...])
    sl = jnp.dot(hs, slw2_ref[...], preferred_element_type=f32) + slb2_ref[...]
    o_ref[...] = out + cat + sl * has


def kernel(polygon_center, polygon_type, polygon_on_route, polygon_tl_status,
           polygon_has_speed_limit, polygon_speed_limit, point_position,
           point_vector, point_orientation, polygon_orientation, valid_mask,
           pe_W1, pe_b1, pe_W2, pe_b2, pe_W3, pe_b3, pe_W4, pe_b4,
           sl_W1, sl_b1, sl_W2, sl_b2, type_emb, on_route_emb, tl_emb, unk_emb):
    f32 = jnp.float32
    pp = point_position.reshape(N, 3 * P * 2)
    pv = point_vector.reshape(N, 3 * P * 2)
    po = point_orientation.reshape(N, 3 * P)
    ctr = polygon_center.reshape(N, 3)

    grid = (N // BLK,)
    row = lambda shape: pl.BlockSpec(shape, lambda i: (i, 0))
    rep = lambda shape: pl.BlockSpec(shape, lambda i: (0, 0))
    out = pl.pallas_call(
        _fused_kernel,
        grid=grid,
        in_specs=[
            row((BLK, 3 * P * 2)), row((BLK, 3 * P * 2)), row((BLK, 3 * P)),
            row((BLK, 3)), row((BLK, P)),
            row((BLK, 1)), row((BLK, 1)), row((BLK, 1)), row((BLK, 1)),
            row((BLK, 1)),
            rep((6, 128)), rep((1, 128)),
            rep((128, 256)), rep((1, 256)),
            rep((512, 256)), rep((1, 256)),
            rep((256, 128)), rep((1, 128)),
            rep((1, 128)), rep((1, 128)), rep((128, 128)), rep((1, 128)),
            rep((3, 128)), rep((2, 128)), rep((4, 128)), rep((1, 128)),
        ],
        out_specs=pl.BlockSpec((BLK, DIM), lambda i: (i, 0)),
        out_shape=jax.ShapeDtypeStruct((N, DIM), f32),
        scratch_shapes=[
            pltpu.VMEM((128, P * 128), f32),
            pltpu.VMEM((128, 256), jnp.bfloat16),
            pltpu.VMEM((256, 256), jnp.bfloat16),
            pltpu.VMEM((256, 256), jnp.bfloat16),
            pltpu.VMEM((256, 128), jnp.bfloat16),
        ],
    )(pp, pv, po, ctr, valid_mask.reshape(N, P),
      polygon_type.reshape(N, 1), polygon_on_route.reshape(N, 1),
      polygon_tl_status.reshape(N, 1), polygon_has_speed_limit.reshape(N, 1),
      polygon_speed_limit.reshape(N, 1),
      pe_W1, pe_b1.reshape(1, 128), pe_W2, pe_b2.reshape(1, 256),
      pe_W3, pe_b3.reshape(1, 256), pe_W4, pe_b4.reshape(1, 128),
      sl_W1, sl_b1.reshape(1, 128), sl_W2, sl_b2.reshape(1, 128),
      type_emb, on_route_emb, tl_emb, unk_emb)
    return out.reshape(BS, M, DIM)


# speedup vs baseline: 1.1789x; 1.0402x over previous
import jax
import jax.numpy as jnp
from jax.experimental import pallas as pl
from jax.experimental.pallas import tpu as pltpu

BS, M, P, DIM = 32, 128, 20, 128
N = BS * M
BLK = 512
BN_SCALE = 0.9999950000374997


def _fused_kernel(pp_ref, pv_ref, po_ref, ctr_ref, mask_ref,
                  t_ref, r_ref, tl_ref, has_ref, s_ref,
                  w1_ref, b1_ref, w2_ref, b2_ref, w3_ref, b3_ref,
                  w4_ref, b4_ref, slw1_ref, slb1_ref, slw2_ref, slb2_ref,
                  temb_ref, remb_ref, tlemb_ref, uemb_ref, o_ref,
                  w1c_ref, w2s_ref, w3ts_ref, w3bs_ref, w4s_ref):
    f32 = jnp.float32
    bf16 = jnp.bfloat16
    bn = f32(BN_SCALE)

    @pl.when(pl.program_id(0) == 0)
    def _prep():
        w2s_ref[...] = (w2_ref[...] * bn).astype(bf16)
        w3ts_ref[...] = w3_ref[0:256].astype(bf16)
        w3bs_ref[...] = w3_ref[256:512].astype(bf16)
        w4s_ref[...] = (w4_ref[...] * bn).astype(bf16)
        w1c_ref[...] = jnp.zeros((128, P * 128), f32)
        for p in range(P):
            c = 128 * p
            w1c_ref[2 * p:2 * p + 2, c:c + 128] = w1_ref[0:2]
            w1c_ref[2 * P + 2 * p:2 * P + 2 * p + 2, c:c + 128] = w1_ref[2:4]
            w1c_ref[4 * P + p:4 * P + p + 1, c:c + 128] = w1_ref[4:5]
            w1c_ref[5 * P + p:5 * P + p + 1, c:c + 128] = w1_ref[5:6]
            w1c_ref[6 * P:6 * P + 2, c:c + 128] = -w1_ref[0:2]

    b1 = b1_ref[...]
    w2 = w2s_ref[...]
    b2 = b2_ref[...]
    mask = mask_ref[...].astype(f32)

    po = po_ref[:, 0:P]
    lhs = jnp.concatenate(
        [pp_ref[:, 0:2 * P], pv_ref[:, 0:2 * P], jnp.cos(po), jnp.sin(po),
         ctr_ref[:, 0:2], jnp.zeros((BLK, 128 - 6 * P - 2), f32)], axis=1)
    h1a = jnp.dot(lhs, w1c_ref[...], preferred_element_type=f32)

    h2s = []
    pooled = None
    for p in range(P):
        h1 = jax.nn.relu(h1a[:, 128 * p:128 * p + 128] + b1)
        h2 = jnp.dot(h1.astype(bf16), w2, preferred_element_type=f32) + b2
        h2 = (h2 * mask[:, p:p + 1]).astype(bf16)
        h2s.append(h2)
        pooled = h2 if pooled is None else jnp.maximum(pooled, h2)

    w3t = w3ts_ref[...]
    pb = jnp.dot(pooled, w3bs_ref[...], preferred_element_type=f32) + b3_ref[...]

    w4 = w4s_ref[...]
    b4 = b4_ref[...]
    out = None
    for p in range(P):
        g1 = jax.nn.relu(jnp.dot(h2s[p], w3t, preferred_element_type=f32) + pb)
        g = jnp.dot(g1.astype(bf16), w4, preferred_element_type=f32) + b4
        g = g * mask[:, p:p + 1]
        out = g if out is None else jnp.maximum(out, g)

    has = has_ref[...].astype(f32)
    iota = jax.lax.broadcasted_iota(jnp.int32, (BLK, 16), 1)
    onehot = ((iota == t_ref[...]).astype(f32)
              + (iota == r_ref[...] + 3).astype(f32)
              + (iota == tl_ref[...] + 5).astype(f32)
              + (iota == 9).astype(f32) * (1.0 - has))
    emb = jnp.concatenate(
        [temb_ref[...], remb_ref[...], tlemb_ref[...], uemb_ref[...],
         jnp.zeros((6, DIM), f32)], axis=0)
    cat = jnp.dot(onehot, emb, preferred_element_type=f32)

    hs = jax.nn.relu(s_ref[...] * slw1_ref[...] + slb1_ref[...])
    sl = jnp.dot(hs, slw2_ref[...], preferred_element_type=f32) + slb2_ref[...]
    o_ref[...] = out + cat + sl * has


def kernel(polygon_center, polygon_type, polygon_on_route, polygon_tl_status,
           polygon_has_speed_limit, polygon_speed_limit, point_position,
           point_vector, point_orientation, polygon_orientation, valid_mask,
           pe_W1, pe_b1, pe_W2, pe_b2, pe_W3, pe_b3, pe_W4, pe_b4,
           sl_W1, sl_b1, sl_W2, sl_b2, type_emb, on_route_emb, tl_emb, unk_emb):
    f32 = jnp.float32
    pp = point_position.reshape(N, 3 * P * 2)
    pv = point_vector.reshape(N, 3 * P * 2)
    po = point_orientation.reshape(N, 3 * P)
    ctr = polygon_center.reshape(N, 3)

    grid = (N // BLK,)
    row = lambda shape: pl.BlockSpec(shape, lambda i: (i, 0))
    rep = lambda shape: pl.BlockSpec(shape, lambda i: (0, 0))
    out = pl.pallas_call(
        _fused_kernel,
        grid=grid,
        in_specs=[
            row((BLK, 3 * P * 2)), row((BLK, 3 * P * 2)), row((BLK, 3 * P)),
            row((BLK, 3)), row((BLK, P)),
            row((BLK, 1)), row((BLK, 1)), row((BLK, 1)), row((BLK, 1)),
            row((BLK, 1)),
            rep((6, 128)), rep((1, 128)),
            rep((128, 256)), rep((1, 256)),
            rep((512, 256)), rep((1, 256)),
            rep((256, 128)), rep((1, 128)),
            rep((1, 128)), rep((1, 128)), rep((128, 128)), rep((1, 128)),
            rep((3, 128)), rep((2, 128)), rep((4, 128)), rep((1, 128)),
        ],
        out_specs=pl.BlockSpec((BLK, DIM), lambda i: (i, 0)),
        out_shape=jax.ShapeDtypeStruct((N, DIM), f32),
        scratch_shapes=[
            pltpu.VMEM((128, P * 128), f32),
            pltpu.VMEM((128, 256), jnp.bfloat16),
            pltpu.VMEM((256, 256), jnp.bfloat16),
            pltpu.VMEM((256, 256), jnp.bfloat16),
            pltpu.VMEM((256, 128), jnp.bfloat16),
        ],
    )(pp, pv, po, ctr, valid_mask.reshape(N, P),
      polygon_type.reshape(N, 1), polygon_on_route.reshape(N, 1),
      polygon_tl_status.reshape(N, 1), polygon_has_speed_limit.reshape(N, 1),
      polygon_speed_limit.reshape(N, 1),
      pe_W1, pe_b1.reshape(1, 128), pe_W2, pe_b2.reshape(1, 256),
      pe_W3, pe_b3.reshape(1, 256), pe_W4, pe_b4.reshape(1, 128),
      sl_W1, sl_b1.reshape(1, 128), sl_W2, sl_b2.reshape(1, 128),
      type_emb, on_route_emb, tl_emb, unk_emb)
    return out.reshape(BS, M, DIM)
